# baseline (speedup 1.0000x reference)
import functools

import jax
import jax.numpy as jnp
from jax import lax
from jax.experimental import pallas as pl
from jax.experimental.pallas import tpu as pltpu
from jax.experimental.pallas import tpu_sc as plsc

B, T, H = 32, 16, 768
E, K, F = 64, 8, 256
N = B * T
EPB = 8
_NEG = -3.0e38

_NC, _NS, _L = 2, 16, 16
_NW = _NC * _NS
_TPW = N // _NW
_Q = E // _L


def _logits_body(x_ref, rw_ref, logits_ref):
    logits_ref[...] = jnp.dot(x_ref[...], rw_ref[...],
                              preferred_element_type=jnp.float32)


def _tree(op, x):
    n = x.shape[1]
    while n > 1:
        n //= 2
        x = op(x[:, 0:n], x[:, n:2 * n])
    return x


def _sc_combine_body(logits_hbm, col_hbm, comb_hbm, lv, cv, colv):
    wid = lax.axis_index("s") * _NC + lax.axis_index("c")
    base = wid * _TPW
    pltpu.sync_copy(logits_hbm.at[pl.ds(base, _TPW)], lv)
    pltpu.sync_copy(col_hbm, colv)
    a = lv[...]
    colm = colv[...]
    remaining = a
    mtop = None
    picked = jnp.zeros((_TPW, E), dtype=jnp.bool_)
    for _ in range(K):
        mk = _tree(jnp.maximum, remaining)
        if mtop is None:
            mtop = mk
        is_max = remaining == mk
        first = _tree(jnp.minimum, jnp.where(is_max, colm, E))
        sel = colm == first
        picked = jnp.logical_or(picked, sel)
        remaining = jnp.where(sel, _NEG, remaining)
    s = jnp.where(picked, jnp.exp(a - mtop), 0.0)
    cv[...] = s / _tree(jnp.add, s)
    pltpu.sync_copy(cv, comb_hbm.at[pl.ds(base, _TPW)])


def _ffn_body(x_ref, comb_ref, wg_ref, wu_ref, wd_ref, out_ref, xb_ref):
    i = pl.program_id(0)

    @pl.when(i == 0)
    def _():
        xb_ref[...] = x_ref[...].astype(jnp.bfloat16)

    xb = xb_ref[...]
    acc = None
    for j in range(EPB):
        wg = wg_ref[j].astype(jnp.bfloat16)
        wu = wu_ref[j].astype(jnp.bfloat16)
        g = jnp.dot(xb, wg, preferred_element_type=jnp.float32)
        u = jnp.dot(xb, wu, preferred_element_type=jnp.float32)
        a = (g * jax.nn.sigmoid(g)) * u
        col = jax.lax.broadcasted_iota(jnp.int32, (N, E), 1)
        c = jnp.sum(jnp.where(col == i * EPB + j, comb_ref[...], 0.0),
                    axis=1, keepdims=True)
        wd = wd_ref[j].astype(jnp.bfloat16)
        y = jnp.dot((c * a).astype(jnp.bfloat16), wd,
                    preferred_element_type=jnp.float32)
        acc = y if acc is None else acc + y

    @pl.when(i == 0)
    def _():
        out_ref[...] = acc

    @pl.when(i != 0)
    def _():
        out_ref[...] += acc


@functools.partial(jax.jit, static_argnames=())
def kernel(x, router_w, w_gate, w_up, w_down):
    flat = x.reshape(N, H)
    logits = pl.pallas_call(
        _logits_body,
        out_shape=jax.ShapeDtypeStruct((N, E), jnp.float32),
    )(flat, router_w)

    sc_combine = pl.kernel(
        _sc_combine_body,
        out_type=jax.ShapeDtypeStruct((N, E), jnp.float32),
        mesh=plsc.VectorSubcoreMesh(core_axis_name="c", subcore_axis_name="s"),
        scratch_types=[
            pltpu.VMEM((_TPW, E), jnp.float32),
            pltpu.VMEM((_TPW, E), jnp.float32),
            pltpu.VMEM((_TPW, E), jnp.int32),
        ],
    )
    colmat = lax.broadcasted_iota(jnp.int32, (_TPW, E), 1)
    comb = sc_combine(logits, colmat)

    out = pl.pallas_call(
        _ffn_body,
        grid=(E // EPB,),
        in_specs=[
            pl.BlockSpec((N, H), lambda i: (0, 0)),
            pl.BlockSpec((N, E), lambda i: (0, 0)),
            pl.BlockSpec((EPB, H, F), lambda i: (i, 0, 0)),
            pl.BlockSpec((EPB, H, F), lambda i: (i, 0, 0)),
            pl.BlockSpec((EPB, F, H), lambda i: (i, 0, 0)),
        ],
        out_specs=pl.BlockSpec((N, H), lambda i: (0, 0)),
        out_shape=jax.ShapeDtypeStruct((N, H), jnp.float32),
        scratch_shapes=[
            pltpu.VMEM((N, H), jnp.bfloat16),
        ],
    )(flat, comb, w_gate, w_up, w_down)

    return out.reshape(B, T, H), logits
